# baseline (speedup 1.0000x reference)
import functools

import jax
import jax.numpy as jnp
from jax import lax
from jax.experimental import pallas as pl
from jax.experimental.pallas import tpu as pltpu
from jax.experimental.pallas import tpu_sc as plsc

_N_TOKENS = 32768
_D_MODEL = 2048
_NUM_EXPERTS = 64
_TOP_K = 8
_BT = 1024
_NBUF = 4
_NW = 32
_C = 4
_TWC = _N_TOKENS // _C // _NW
_G = _BT // _TWC


def _mm_body(off, x_hbm, w_ref, o_ref, xbuf, sems):
    i = pl.program_id(0)
    nblk = pl.num_programs(0)

    def cp(blk, slot):
        return pltpu.make_async_copy(
            x_hbm.at[pl.ds((off + blk) * _BT, _BT), :], xbuf.at[slot],
            sems.at[slot])

    @pl.when(i == 0)
    def _prime():
        for b in range(_NBUF - 1):
            cp(b, b).start()

    nxt = i + _NBUF - 1
    @pl.when(nxt < nblk)
    def _prefetch():
        cp(nxt, nxt % _NBUF).start()

    slot = i % _NBUF
    cp(i, slot).wait()
    vals = jax.lax.dot_general(
        w_ref[...], xbuf[slot], (((1,), (1,)), ((), ())),
        preferred_element_type=jnp.float32,
    )
    for j in range(_G):
        o_ref[j] = vals[:, j * _TWC:(j + 1) * _TWC]


def _tc_logits(x, W, c):
    nblk = _N_TOKENS // _C // _BT
    return pl.pallas_call(
        functools.partial(_mm_body, c * nblk),
        grid=(nblk,),
        in_specs=[
            pl.BlockSpec(memory_space=pl.ANY),
            pl.BlockSpec((_NUM_EXPERTS, _D_MODEL), lambda i: (0, 0)),
        ],
        out_specs=pl.BlockSpec((_G, _NUM_EXPERTS, _TWC), lambda i: (i, 0, 0)),
        out_shape=jax.ShapeDtypeStruct((_NW, _NUM_EXPERTS, _TWC), jnp.float32),
        scratch_shapes=[
            pltpu.VMEM((_NBUF, _BT, _D_MODEL), jnp.float32),
            pltpu.SemaphoreType.DMA((_NBUF,)),
        ],
    )(x, W)


@functools.partial(
    pl.kernel,
    mesh=plsc.VectorSubcoreMesh(core_axis_name="c", subcore_axis_name="s"),
    out_type=[
        jax.ShapeDtypeStruct((_NW, _TOP_K, _TWC), jnp.float32),
        jax.ShapeDtypeStruct((_NW, _TOP_K, _TWC), jnp.int32),
    ],
    scratch_types=[
        pltpu.VMEM((_NUM_EXPERTS, _TWC), jnp.float32),
        pltpu.VMEM((_TOP_K, _TWC), jnp.float32),
        pltpu.VMEM((_TOP_K, _TWC), jnp.int32),
    ],
)
def _sc_topk(lg_hbm, ow_hbm, oi_hbm, lbuf, wbuf, ibuf):
    wid = lax.axis_index("s") * 2 + lax.axis_index("c")
    pltpu.sync_copy(lg_hbm.at[wid], lbuf)

    def group(g, carry):
        rv = [jnp.full((16,), -jnp.inf, jnp.float32) for _ in range(_TOP_K)]
        ri = [jnp.zeros((16,), jnp.int32) for _ in range(_TOP_K)]
        for e in range(_NUM_EXPERTS):
            v = lbuf[e, pl.ds(g * 16, 16)]
            ev = jnp.full((16,), e, jnp.int32)
            c = [v > rv[k] for k in range(_TOP_K)]
            nrv = [jnp.where(c[0], v, rv[0])]
            nri = [jnp.where(c[0], ev, ri[0])]
            for k in range(1, _TOP_K):
                nrv.append(jnp.where(c[k], jnp.where(c[k - 1], rv[k - 1], v),
                                     rv[k]))
                nri.append(jnp.where(c[k], jnp.where(c[k - 1], ri[k - 1], ev),
                                     ri[k]))
            rv, ri = nrv, nri
        ex = [jnp.exp(rv[k] - rv[0]) for k in range(_TOP_K)]
        s = ex[0]
        for k in range(1, _TOP_K):
            s = s + ex[k]
        sl = pl.ds(g * 16, 16)
        for k in range(_TOP_K):
            wbuf[k, sl] = ex[k] / s
            ibuf[k, sl] = ri[k]
        return carry

    lax.fori_loop(0, _TWC // 16, group, 0)
    pltpu.sync_copy(wbuf, ow_hbm.at[wid])
    pltpu.sync_copy(ibuf, oi_hbm.at[wid])


@jax.jit
def kernel(x, W):
    ws, ids = [], []
    for c in range(_C):
        lgt = _tc_logits(x, W, c)
        wf, idf = _sc_topk(lgt)
        ws.append(wf.transpose(0, 2, 1).reshape(-1, _TOP_K))
        ids.append(idf.transpose(0, 2, 1).reshape(-1, _TOP_K))
    return (jnp.concatenate(ws, axis=0), jnp.concatenate(ids, axis=0))


# speedup vs baseline: 1.1853x; 1.1853x over previous
import jax
import jax.numpy as jnp
from jax.experimental import pallas as pl
from jax.experimental.pallas import tpu as pltpu

_N_TOKENS = 32768
_D_MODEL = 2048
_NUM_EXPERTS = 64
_TOP_K = 8
_BT = 1024
_NBUF = 4
_NBLK = _N_TOKENS // _BT


def _top8_softmax(vals, out_w_ref, out_i_ref):
    iota = jax.lax.broadcasted_iota(jnp.int32, vals.shape, 0)
    top_vals = []
    top_idxs = []
    for _ in range(_TOP_K):
        m = jnp.max(vals, axis=0, keepdims=True)
        idx = jnp.min(jnp.where(vals == m, iota, _NUM_EXPERTS), axis=0,
                      keepdims=True)
        top_vals.append(m)
        top_idxs.append(idx)
        vals = jnp.where(iota == idx, -jnp.inf, vals)
    tv = jnp.concatenate(top_vals, axis=0)
    ti = jnp.concatenate(top_idxs, axis=0)
    e = jnp.exp(tv - tv[0:1])
    wgt = e / jnp.sum(e, axis=0, keepdims=True)
    eye = jnp.eye(_TOP_K, dtype=jnp.float32)
    out_w_ref[...] = jax.lax.dot_general(
        wgt, eye, (((0,), (0,)), ((), ())),
        preferred_element_type=jnp.float32)
    ti_f = ti.astype(jnp.float32)
    out_i_ref[...] = jax.lax.dot_general(
        ti_f, eye, (((0,), (0,)), ((), ())),
        preferred_element_type=jnp.float32).astype(jnp.int32)


def _gate_body(x_hbm, w_ref, ow_ref, oi_ref, xbuf, lbuf, sems):
    i = pl.program_id(0)

    def cp(blk, slot):
        return pltpu.make_async_copy(
            x_hbm.at[pl.ds(blk * _BT, _BT), :], xbuf.at[slot], sems.at[slot])

    @pl.when(i == 0)
    def _prime():
        for b in range(_NBUF - 1):
            cp(b, b).start()

    nxt = i + _NBUF - 1
    @pl.when(nxt < _NBLK)
    def _prefetch():
        cp(nxt, nxt % _NBUF).start()

    @pl.when(i < _NBLK)
    def _mm():
        slot = i % _NBUF
        cp(i, slot).wait()
        lbuf[i % 2] = jax.lax.dot_general(
            w_ref[...], xbuf[slot], (((1,), (1,)), ((), ())),
            preferred_element_type=jnp.float32,
        )

    @pl.when(i > 0)
    def _route():
        _top8_softmax(lbuf[(i - 1) % 2], ow_ref, oi_ref)


@jax.jit
def kernel(x, W):
    grid = (_NBLK + 1,)
    return pl.pallas_call(
        _gate_body,
        grid=grid,
        in_specs=[
            pl.BlockSpec(memory_space=pl.ANY),
            pl.BlockSpec((_NUM_EXPERTS, _D_MODEL), lambda i: (0, 0)),
        ],
        out_specs=[
            pl.BlockSpec((_BT, _TOP_K), lambda i: (jnp.maximum(i - 1, 0), 0)),
            pl.BlockSpec((_BT, _TOP_K), lambda i: (jnp.maximum(i - 1, 0), 0)),
        ],
        out_shape=[
            jax.ShapeDtypeStruct((_N_TOKENS, _TOP_K), jnp.float32),
            jax.ShapeDtypeStruct((_N_TOKENS, _TOP_K), jnp.int32),
        ],
        scratch_shapes=[
            pltpu.VMEM((_NBUF, _BT, _D_MODEL), jnp.float32),
            pltpu.VMEM((2, _NUM_EXPERTS, _BT), jnp.float32),
            pltpu.SemaphoreType.DMA((_NBUF,)),
        ],
    )(x, W)
